# baseline (speedup 1.0000x reference)
import jax
import jax.numpy as jnp
from jax.experimental import pallas as pl
from jax.experimental.pallas import tpu as pltpu

D_MODEL = 1024
SEQ = 4
PERIOD = 8
GROUPS_PER_BLOCK = 16


def _body(x_ref, e2_ref, e4_ref, e8_ref, o_ref):
    t = (
        jnp.tile(e2_ref[...], (4, 1))
        + jnp.tile(e4_ref[...], (2, 1))
        + e8_ref[...]
    )
    add = jnp.tile(t, (1, SEQ))
    o_ref[...] = x_ref[...] + add[None, :, :]


def kernel(x, emb2, emb4, emb8):
    L = x.shape[0]
    g = L // PERIOD
    xv = x.reshape(g, PERIOD, SEQ * D_MODEL)
    grid = (g // GROUPS_PER_BLOCK,)
    out = pl.pallas_call(
        _body,
        grid=grid,
        in_specs=[
            pl.BlockSpec((GROUPS_PER_BLOCK, PERIOD, SEQ * D_MODEL), lambda i: (i, 0, 0)),
            pl.BlockSpec((2, D_MODEL), lambda i: (0, 0)),
            pl.BlockSpec((4, D_MODEL), lambda i: (0, 0)),
            pl.BlockSpec((8, D_MODEL), lambda i: (0, 0)),
        ],
        out_specs=pl.BlockSpec((GROUPS_PER_BLOCK, PERIOD, SEQ * D_MODEL), lambda i: (i, 0, 0)),
        out_shape=jax.ShapeDtypeStruct((g, PERIOD, SEQ * D_MODEL), x.dtype),
        compiler_params=pltpu.CompilerParams(
            dimension_semantics=("parallel",),
        ),
    )(xv, emb2, emb4, emb8)
    return out.reshape(L, SEQ, D_MODEL)


# speedup vs baseline: 4.6539x; 4.6539x over previous
import jax
import jax.numpy as jnp
from jax.experimental import pallas as pl
from jax.experimental.pallas import tpu as pltpu

D_MODEL = 1024
SEQ = 4
PERIOD = 8
ROWS_PER_BLOCK = 256


def _body(x_ref, e2_ref, e4_ref, e8_ref, o_ref):
    t = (
        jnp.tile(e2_ref[...], (4, 1, 1))
        + jnp.tile(e4_ref[...], (2, 1, 1))
        + e8_ref[...]
    )
    add = jnp.tile(t, (ROWS_PER_BLOCK // PERIOD, 1, 1))
    o_ref[...] = x_ref[...] + add


def kernel(x, emb2, emb4, emb8):
    L = x.shape[0]
    grid = (L // ROWS_PER_BLOCK,)
    return pl.pallas_call(
        _body,
        grid=grid,
        in_specs=[
            pl.BlockSpec((ROWS_PER_BLOCK, SEQ, D_MODEL), lambda i: (i, 0, 0)),
            pl.BlockSpec((2, 1, D_MODEL), lambda i: (0, 0, 0)),
            pl.BlockSpec((4, 1, D_MODEL), lambda i: (0, 0, 0)),
            pl.BlockSpec((8, 1, D_MODEL), lambda i: (0, 0, 0)),
        ],
        out_specs=pl.BlockSpec((ROWS_PER_BLOCK, SEQ, D_MODEL), lambda i: (i, 0, 0)),
        out_shape=jax.ShapeDtypeStruct((L, SEQ, D_MODEL), x.dtype),
        compiler_params=pltpu.CompilerParams(
            dimension_semantics=("parallel",),
        ),
    )(x, emb2[:, None, :], emb4[:, None, :], emb8[:, None, :])
